# speedup vs baseline: 5.5590x; 5.5590x over previous
import functools

import jax
import jax.numpy as jnp
from jax import lax
from jax.experimental import pallas as pl
from jax.experimental.pallas import tpu as pltpu
from jax.experimental.pallas import tpu_sc as plsc

N = 16384
C = 128
K = 9
OUT = 128
GRID_SIZE = C
TAB_PAD = 320

R = 256
NB = N // R

SC_WORKERS = 32
PER_W = N // SC_WORKERS
CH = 256


def _knn_body(xr_ref, xf_ref, tab_ref, nbr_ref, xp_ref):
    i = pl.program_id(0)
    xr = xr_ref[...]
    xf = xf_ref[...]
    x2f = jnp.sum(xf * xf, axis=1)
    x2r = jnp.sum(xr * xr, axis=1)
    s = lax.dot_general(xr, xf, (((1,), (1,)), ((), ())),
                        preferred_element_type=jnp.float32)
    d = (x2r[:, None] + x2f[None, :]) - 2.0 * s
    rows = i * R + lax.broadcasted_iota(jnp.int32, (R, N), 0)
    cols = lax.broadcasted_iota(jnp.int32, (R, N), 1)
    d = jnp.where(cols == rows, jnp.inf, d)
    for k in range(K):
        m = jnp.min(d, axis=1)
        eq = d == m[:, None]
        am = jnp.min(jnp.where(eq, cols, N), axis=1)
        nbr_ref[k, :] = am
        if k + 1 < K:
            d = jnp.where(eq, jnp.inf, d)
    for k in range(K, 16):
        nbr_ref[k, :] = jnp.zeros((R,), jnp.int32)
    rid = i * R + lax.iota(jnp.int32, R)
    rel = rid // GRID_SIZE - rid % GRID_SIZE + (GRID_SIZE - 1)
    onehot = (rel[:, None] ==
              lax.broadcasted_iota(jnp.int32, (R, TAB_PAD), 1)).astype(jnp.float32)
    emb = jnp.dot(onehot, tab_ref[...], preferred_element_type=jnp.float32)
    xp_ref[...] = xr + emb


def _knn_pallas(x, tab):
    return pl.pallas_call(
        _knn_body,
        grid=(NB,),
        in_specs=[
            pl.BlockSpec((R, C), lambda i: (i, 0)),
            pl.BlockSpec((N, C), lambda i: (0, 0)),
            pl.BlockSpec((TAB_PAD, C), lambda i: (0, 0)),
        ],
        out_specs=[
            pl.BlockSpec((16, R), lambda i: (0, i)),
            pl.BlockSpec((R, C), lambda i: (i, 0)),
        ],
        out_shape=[
            jax.ShapeDtypeStruct((16, N), jnp.int32),
            jax.ShapeDtypeStruct((N, C), jnp.float32),
        ],
        compiler_params=pltpu.CompilerParams(
            dimension_semantics=("arbitrary",)),
    )(x, x, tab)


def _aggmax_body(xp_hbm, nb_hbm, out_hbm, idx_v, rows_v, acc_v, sem):
    wid = lax.axis_index("s") * 2 + lax.axis_index("c")
    for c2 in range(PER_W // CH):
        base = wid * PER_W + c2 * CH
        pltpu.sync_copy(nb_hbm.at[0, pl.ds(base, CH)], idx_v)
        pltpu.async_copy(xp_hbm.at[idx_v], acc_v, sem).wait()
        for k in range(1, K):
            pltpu.sync_copy(nb_hbm.at[k, pl.ds(base, CH)], idx_v)
            pltpu.async_copy(xp_hbm.at[idx_v], rows_v, sem).wait()

            def maxbody(r, carry):
                for j in range(C // 16):
                    sl = pl.ds(j * 16, 16)
                    acc_v[r, sl] = jnp.maximum(acc_v[r, sl], rows_v[r, sl])
                return carry

            lax.fori_loop(0, CH, maxbody, 0)
        pltpu.sync_copy(acc_v, out_hbm.at[pl.ds(base, CH)])


def _aggmax_sc(xprime, nbrT):
    mesh = plsc.VectorSubcoreMesh(core_axis_name="c", subcore_axis_name="s")
    fn = functools.partial(
        pl.kernel,
        mesh=mesh,
        out_type=jax.ShapeDtypeStruct((N, C), jnp.float32),
        scratch_types=[
            pltpu.VMEM((CH,), jnp.int32),
            pltpu.VMEM((CH, C), jnp.float32),
            pltpu.VMEM((CH, C), jnp.float32),
            pltpu.SemaphoreType.DMA,
        ],
    )(_aggmax_body)
    return fn(xprime, nbrT)


def _out_body(xp_ref, ag_ref, w_ref, b_ref, o_ref):
    w1 = w_ref[0:C, :]
    w2 = w_ref[C:2 * C, :]
    o_ref[...] = (jnp.dot(xp_ref[...], w1 - w2, preferred_element_type=jnp.float32)
                  + jnp.dot(ag_ref[...], w2, preferred_element_type=jnp.float32)
                  + b_ref[...])


def _out_pallas(xprime, aggmax, W, b):
    return pl.pallas_call(
        _out_body,
        grid=(NB,),
        in_specs=[
            pl.BlockSpec((R, C), lambda i: (i, 0)),
            pl.BlockSpec((R, C), lambda i: (i, 0)),
            pl.BlockSpec((2 * C, OUT), lambda i: (0, 0)),
            pl.BlockSpec((1, OUT), lambda i: (0, 0)),
        ],
        out_specs=pl.BlockSpec((R, OUT), lambda i: (i, 0)),
        out_shape=jax.ShapeDtypeStruct((N, OUT), jnp.float32),
        compiler_params=pltpu.CompilerParams(
            dimension_semantics=("arbitrary",)),
    )(xprime, aggmax, W, b.reshape(1, OUT))


def kernel(x, rel_pos_table, W, b):
    tab = jnp.zeros((TAB_PAD, C), jnp.float32).at[:rel_pos_table.shape[0]].set(
        rel_pos_table)
    nbrT, xprime = _knn_pallas(x, tab)
    aggmax = _aggmax_sc(xprime, nbrT)
    return _out_pallas(xprime, aggmax, W, b)
